# baseline (speedup 1.0000x reference)
import functools

import jax
import jax.numpy as jnp
from jax import lax
from jax.experimental import pallas as pl
from jax.experimental.pallas import tpu as pltpu
from jax.experimental.pallas import tpu_sc as plsc

N_NODES = 10000
N_EDGES = 160000
D = 256
L = 4
ATOM_CLASSES = 100
BOND_CLASSES = 20
N_GRAPHS = 64

NC, NS = 2, 16
HALF = N_NODES // NC

G_CHUNK = 128
G_PER_TILE = N_EDGES // (NC * NS)
G_FULL = G_PER_TILE // G_CHUNK
G_TAIL = G_PER_TILE - G_FULL * G_CHUNK


def _sc_gather_body(table, src_hbm, tgt_hbm, src_out, tgt_out,
                    idx_v, rows_v, idx_t, rows_t, sem):
    c = lax.axis_index("c")
    s = lax.axis_index("s")
    base = (s * NC + c) * G_PER_TILE

    def one(idx_hbm, out_hbm, off, idx_buf, row_buf, n):
        pltpu.sync_copy(idx_hbm.at[pl.ds(off, n)], idx_buf)
        pltpu.async_copy(table.at[idx_buf], row_buf, sem).wait()
        pltpu.sync_copy(row_buf, out_hbm.at[pl.ds(off, n), :])

    def step(i, carry):
        off = base + i * G_CHUNK
        one(src_hbm, src_out, off, idx_v, rows_v, G_CHUNK)
        one(tgt_hbm, tgt_out, off, idx_v, rows_v, G_CHUNK)
        return carry

    lax.fori_loop(0, G_FULL, step, 0)
    off = base + G_FULL * G_CHUNK
    one(src_hbm, src_out, off, idx_t, rows_t, G_TAIL)
    one(tgt_hbm, tgt_out, off, idx_t, rows_t, G_TAIL)


@jax.jit
def _sc_gather(table, src_idx, tgt_idx):
    mesh = plsc.VectorSubcoreMesh(core_axis_name="c", subcore_axis_name="s")
    f = pl.kernel(
        _sc_gather_body,
        out_type=(jax.ShapeDtypeStruct((N_EDGES, D), jnp.float32),
                  jax.ShapeDtypeStruct((N_EDGES, D), jnp.float32)),
        mesh=mesh,
        scratch_types=[
            pltpu.VMEM((G_CHUNK,), jnp.int32),
            pltpu.VMEM((G_CHUNK, D), jnp.float32),
            pltpu.VMEM((G_TAIL,), jnp.int32),
            pltpu.VMEM((G_TAIL, D), jnp.float32),
            pltpu.SemaphoreType.DMA,
        ],
    )
    return f(table, src_idx, tgt_idx)


S_CHUNK = 1280
S_STEPS = N_EDGES // S_CHUNK
ACC_COLS = 5120
CW = 16
OHP = ACC_COLS


def _splat(v, j):
    lane = jnp.sum(jnp.where(lax.iota(jnp.int32, 16) == j, v, 0))
    return jnp.broadcast_to(lane, (16,)).astype(v.dtype)


def _sc_scatter_body(msgT_hbm, tgt_hbm, zeros_hbm, out_hbm,
                     acc, idx_v, buf, sem):
    c = lax.axis_index("c")
    s = lax.axis_index("s")
    pltpu.sync_copy(zeros_hbm, acc)
    nbase = c * HALF
    iota = lax.iota(jnp.int32, 16)
    dummy = HALF + (iota & 7)
    rowoff = iota * S_CHUNK
    accoff = iota * ACC_COLS

    def step(k, carry):
        off = k * S_CHUNK
        pltpu.sync_copy(tgt_hbm.at[pl.ds(off, S_CHUNK)], idx_v)
        copies = [
            pltpu.make_async_copy(
                msgT_hbm.at[s * CW + f, pl.ds(off, S_CHUNK)],
                buf.at[pl.ds(f * S_CHUNK, S_CHUNK)], sem)
            for f in range(CW)
        ]
        for d in copies:
            d.start()
        for d in copies:
            d.wait()

        def grp(g, carry2):
            tv = idx_v[pl.ds(g * 16, 16)]
            lv = tv - nbase
            ok = (lv >= 0) & (lv < HALF)
            lv = jnp.where(ok, lv, dummy)
            for j in range(16):
                col = jnp.broadcast_to(g * 16 + j, (16,)).astype(jnp.int32)
                val = plsc.load_gather(buf, [rowoff + col])
                plsc.addupdate_scatter(acc, [accoff + _splat(lv, j)], val)
            return carry2

        lax.fori_loop(0, S_CHUNK // 16, grp, 0)
        return carry

    lax.fori_loop(0, S_STEPS, step, 0)
    for f in range(CW):
        pltpu.sync_copy(acc.at[pl.ds(f * ACC_COLS, ACC_COLS)],
                        out_hbm.at[s * CW + f, pl.ds(c * OHP, ACC_COLS)])


@jax.jit
def _sc_scatter(msgT, tgt_idx, zeros):
    mesh = plsc.VectorSubcoreMesh(core_axis_name="c", subcore_axis_name="s")
    f = pl.kernel(
        _sc_scatter_body,
        out_type=jax.ShapeDtypeStruct((D, 2 * OHP), jnp.float32),
        mesh=mesh,
        compiler_params=pltpu.CompilerParams(needs_layout_passes=False),
        scratch_types=[
            pltpu.VMEM((CW * ACC_COLS,), jnp.float32),
            pltpu.VMEM((S_CHUNK,), jnp.int32),
            pltpu.VMEM((CW * S_CHUNK,), jnp.float32),
            pltpu.SemaphoreType.DMA,
        ],
    )
    o = f(msgT, tgt_idx, zeros)
    return jnp.concatenate([o[:, :HALF], o[:, OHP:OHP + HALF]], axis=1).T


BN_BLK = 1000
HI = jax.lax.Precision.HIGHEST


def _node0_body(types_ref, emb_ref, s_ref, t_ref, state_out, abn_out):
    ty = types_ref[:, 0]
    oh = (ty[:, None] == lax.broadcasted_iota(jnp.int32, (BN_BLK, ATOM_CLASSES), 1)
          ).astype(jnp.float32)
    st = jnp.dot(oh, emb_ref[...], precision=HI)
    state_out[...] = st
    abn_out[...] = st * s_ref[...] + t_ref[...]


def _noden_body(prev_ref, agg_ref, s_ref, t_ref, state_out, abn_out):
    st = prev_ref[...] + agg_ref[...]
    state_out[...] = st
    abn_out[...] = st * s_ref[...] + t_ref[...]


def _node0(atom_types2, atom_emb, s_vec, t_vec):
    grid = N_NODES // BN_BLK
    return pl.pallas_call(
        _node0_body,
        grid=(grid,),
        in_specs=[
            pl.BlockSpec((BN_BLK, 1), lambda i: (i, 0)),
            pl.BlockSpec((ATOM_CLASSES, D), lambda i: (0, 0)),
            pl.BlockSpec((1, D), lambda i: (0, 0)),
            pl.BlockSpec((1, D), lambda i: (0, 0)),
        ],
        out_specs=[
            pl.BlockSpec((BN_BLK, D), lambda i: (i, 0)),
            pl.BlockSpec((BN_BLK, D), lambda i: (i, 0)),
        ],
        out_shape=[jax.ShapeDtypeStruct((N_NODES, D), jnp.float32)] * 2,
    )(atom_types2, atom_emb, s_vec, t_vec)


def _noden(prev, agg, s_vec, t_vec):
    grid = N_NODES // BN_BLK
    return pl.pallas_call(
        _noden_body,
        grid=(grid,),
        in_specs=[
            pl.BlockSpec((BN_BLK, D), lambda i: (i, 0)),
            pl.BlockSpec((BN_BLK, D), lambda i: (i, 0)),
            pl.BlockSpec((1, D), lambda i: (0, 0)),
            pl.BlockSpec((1, D), lambda i: (0, 0)),
        ],
        out_specs=[
            pl.BlockSpec((BN_BLK, D), lambda i: (i, 0)),
            pl.BlockSpec((BN_BLK, D), lambda i: (i, 0)),
        ],
        out_shape=[jax.ShapeDtypeStruct((N_NODES, D), jnp.float32)] * 2,
    )(prev, agg, s_vec, t_vec)


BE_BLK = 640


def _edge_core(braw, src_ref, tgt_ref, w1s, w1t, w1b, w2, b2, au, sb, tb,
               msg_out, bond_out):
    sf = src_ref[...].astype(jnp.bfloat16)
    tf = tgt_ref[...].astype(jnp.bfloat16)
    bbn = (braw * sb[...] + tb[...]).astype(jnp.bfloat16)
    acc = (jnp.dot(sf, w1s[...], preferred_element_type=jnp.float32)
           + jnp.dot(tf, w1t[...], preferred_element_type=jnp.float32)
           + jnp.dot(bbn, w1b[...], preferred_element_type=jnp.float32))
    h = jax.nn.sigmoid(acc).astype(jnp.bfloat16)
    nb = jnp.dot(h, w2[...], preferred_element_type=jnp.float32) + b2[...]
    u = jax.nn.sigmoid(jnp.dot(sf, au[...], preferred_element_type=jnp.float32))
    msg_out[...] = (u * nb).T
    if bond_out is not None:
        bond_out[...] = braw + nb


def _edge0_body(btypes_ref, bemb_ref, src_ref, tgt_ref, w1s, w1t, w1b, w2, b2,
                au, sb, tb, msg_out, bond_out):
    ty = btypes_ref[:, 0]
    oh = (ty[:, None] == lax.broadcasted_iota(jnp.int32, (BE_BLK, BOND_CLASSES), 1)
          ).astype(jnp.float32)
    braw = jnp.dot(oh, bemb_ref[...], precision=HI)
    _edge_core(braw, src_ref, tgt_ref, w1s, w1t, w1b, w2, b2, au, sb, tb,
               msg_out, bond_out)


def _edgen_body(bond_ref, src_ref, tgt_ref, w1s, w1t, w1b, w2, b2, au, sb, tb,
                msg_out, bond_out):
    _edge_core(bond_ref[...], src_ref, tgt_ref, w1s, w1t, w1b, w2, b2, au, sb,
               tb, msg_out, bond_out)


def _edge_wspecs():
    return [
        pl.BlockSpec((D, 2 * D), lambda i: (0, 0)),
        pl.BlockSpec((D, 2 * D), lambda i: (0, 0)),
        pl.BlockSpec((D, 2 * D), lambda i: (0, 0)),
        pl.BlockSpec((2 * D, D), lambda i: (0, 0)),
        pl.BlockSpec((1, D), lambda i: (0, 0)),
        pl.BlockSpec((D, D), lambda i: (0, 0)),
        pl.BlockSpec((1, D), lambda i: (0, 0)),
        pl.BlockSpec((1, D), lambda i: (0, 0)),
    ]


def _final_body(state_ref, agg_ref, ngi_ref, w0, b0, w1, b1, wl, bl,
                out_ref, acc):
    i = pl.program_id(0)

    @pl.when(i == 0)
    def _():
        acc[...] = jnp.zeros_like(acc)

    st = state_ref[...] + agg_ref[...]
    g = ngi_ref[:, 0]
    oh = (g[:, None] == lax.broadcasted_iota(jnp.int32, (BN_BLK, N_GRAPHS), 1)
          ).astype(jnp.float32)
    acc[...] += lax.dot_general(oh, st, (((0,), (0,)), ((), ())), precision=HI)

    @pl.when(i == pl.num_programs(0) - 1)
    def _():
        mol = acc[...]
        m = jax.nn.relu(jnp.dot(mol, w0[...], precision=HI) + b0[...])
        m = jax.nn.relu(jnp.dot(m, w1[...], precision=HI) + b1[...])
        out_ref[...] = jnp.dot(m, wl[...], precision=HI) + bl[...]


def _final(state, agg, ngi2, w0, b0, w1, b1, wl, bl):
    grid = N_NODES // BN_BLK
    return pl.pallas_call(
        _final_body,
        grid=(grid,),
        in_specs=[
            pl.BlockSpec((BN_BLK, D), lambda i: (i, 0)),
            pl.BlockSpec((BN_BLK, D), lambda i: (i, 0)),
            pl.BlockSpec((BN_BLK, 1), lambda i: (i, 0)),
            pl.BlockSpec((D, 256), lambda i: (0, 0)),
            pl.BlockSpec((1, 256), lambda i: (0, 0)),
            pl.BlockSpec((256, 128), lambda i: (0, 0)),
            pl.BlockSpec((1, 128), lambda i: (0, 0)),
            pl.BlockSpec((128, 1), lambda i: (0, 0)),
            pl.BlockSpec((1, 1), lambda i: (0, 0)),
        ],
        out_specs=pl.BlockSpec((N_GRAPHS, 1), lambda i: (0, 0)),
        out_shape=jax.ShapeDtypeStruct((N_GRAPHS, 1), jnp.float32),
        scratch_shapes=[pltpu.VMEM((N_GRAPHS, D), jnp.float32)],
    )(state, agg, ngi2, w0, b0, w1, b1, wl, bl)


def kernel(atom_types, bond_types, node_graph_indices, connectivity, atom_emb,
           bond_emb, atom_bn_gamma, atom_bn_beta, atom_bn_mean, atom_bn_var,
           bond_bn_gamma, bond_bn_beta, bond_bn_mean, bond_bn_var, bu1_W,
           bu2_W, bu2_b, au_W, out0_W, out0_b, out1_W, out1_b, last_W, last_b):
    eps = 1e-3
    sa = atom_bn_gamma / jnp.sqrt(atom_bn_var + eps)
    ta = atom_bn_beta - atom_bn_mean * sa
    sb = bond_bn_gamma / jnp.sqrt(bond_bn_var + eps)
    tb = bond_bn_beta - bond_bn_mean * sb

    tgt_idx = connectivity[:, 0]
    src_idx = connectivity[:, 1]
    zeros = jnp.zeros((CW * ACC_COLS,), jnp.float32)

    at2 = atom_types.reshape(N_NODES, 1)
    bt2 = bond_types.reshape(N_EDGES, 1)
    ngi2 = node_graph_indices.reshape(N_NODES, 1)

    w1s = [bu1_W[i, :D].astype(jnp.bfloat16) for i in range(L)]
    w1t = [bu1_W[i, D:2 * D].astype(jnp.bfloat16) for i in range(L)]
    w1b = [bu1_W[i, 2 * D:].astype(jnp.bfloat16) for i in range(L)]
    w2 = [bu2_W[i].astype(jnp.bfloat16) for i in range(L)]
    au = [au_W[i].astype(jnp.bfloat16) for i in range(L)]
    b2 = [bu2_b[i].reshape(1, D) for i in range(L)]
    sbv = [sb[i].reshape(1, D) for i in range(L)]
    tbv = [tb[i].reshape(1, D) for i in range(L)]
    sav = [sa[i].reshape(1, D) for i in range(L)]
    tav = [ta[i].reshape(1, D) for i in range(L)]

    eb = pl.BlockSpec((BE_BLK, D), lambda i: (i, 0))
    grid_e = N_EDGES // BE_BLK

    state, abn = _node0(at2, atom_emb, sav[0], tav[0])
    bond = None
    agg = None
    for i in range(L):
        if i > 0:
            state, abn = _noden(state, agg, sav[i], tav[i])
        src_rows, tgt_rows = _sc_gather(abn, src_idx, tgt_idx)
        first = (i == 0)
        want_bond = (i < L - 1)
        if first:
            lead_args = [bt2, bond_emb]
            lead_specs = [pl.BlockSpec((BE_BLK, 1), lambda i: (i, 0)),
                          pl.BlockSpec((BOND_CLASSES, D), lambda i: (0, 0))]
            body = _edge0_body
        else:
            lead_args = [bond]
            lead_specs = [eb]
            body = _edgen_body
        ebT = pl.BlockSpec((D, BE_BLK), lambda i: (0, i))
        if want_bond:
            fn = body
            out_specs = [ebT, eb]
            out_shape = [jax.ShapeDtypeStruct((D, N_EDGES), jnp.float32),
                         jax.ShapeDtypeStruct((N_EDGES, D), jnp.float32)]
        else:
            def fn(*args, _body=body):
                _body(*args, None)
            out_specs = [ebT]
            out_shape = [jax.ShapeDtypeStruct((D, N_EDGES), jnp.float32)]
        res = pl.pallas_call(
            fn,
            grid=(grid_e,),
            in_specs=lead_specs + [eb, eb] + _edge_wspecs(),
            out_specs=out_specs,
            out_shape=out_shape,
        )(*lead_args, src_rows, tgt_rows, w1s[i], w1t[i], w1b[i], w2[i],
          b2[i], au[i], sbv[i], tbv[i])
        if want_bond:
            msg, bond = res
        else:
            msg, = res
        agg = _sc_scatter(msg, tgt_idx, zeros)

    return _final(state, agg, ngi2,
                  out0_W, out0_b.reshape(1, 256),
                  out1_W, out1_b.reshape(1, 128),
                  last_W, last_b.reshape(1, 1))


# speedup vs baseline: 1.0131x; 1.0131x over previous
import functools

import jax
import jax.numpy as jnp
from jax import lax
from jax.experimental import pallas as pl
from jax.experimental.pallas import tpu as pltpu
from jax.experimental.pallas import tpu_sc as plsc

N_NODES = 10000
N_EDGES = 160000
D = 256
L = 4
ATOM_CLASSES = 100
BOND_CLASSES = 20
N_GRAPHS = 64

NC, NS = 2, 16
HALF = N_NODES // NC

G_CHUNK = 128
G_PER_TILE = N_EDGES // (NC * NS)
G_FULL = G_PER_TILE // G_CHUNK
G_TAIL = G_PER_TILE - G_FULL * G_CHUNK


def _sc_gather_body(table, src_hbm, tgt_hbm, src_out, tgt_out,
                    idx_v, rows_v, idx_t, rows_t, sem):
    c = lax.axis_index("c")
    s = lax.axis_index("s")
    base = (s * NC + c) * G_PER_TILE

    def one(idx_hbm, out_hbm, off, idx_buf, row_buf, n):
        pltpu.sync_copy(idx_hbm.at[pl.ds(off, n)], idx_buf)
        pltpu.async_copy(table.at[idx_buf], row_buf, sem).wait()
        pltpu.sync_copy(row_buf, out_hbm.at[pl.ds(off, n), :])

    def step(i, carry):
        off = base + i * G_CHUNK
        one(src_hbm, src_out, off, idx_v, rows_v, G_CHUNK)
        one(tgt_hbm, tgt_out, off, idx_v, rows_v, G_CHUNK)
        return carry

    lax.fori_loop(0, G_FULL, step, 0)
    off = base + G_FULL * G_CHUNK
    one(src_hbm, src_out, off, idx_t, rows_t, G_TAIL)
    one(tgt_hbm, tgt_out, off, idx_t, rows_t, G_TAIL)


@jax.jit
def _sc_gather(table, src_idx, tgt_idx):
    mesh = plsc.VectorSubcoreMesh(core_axis_name="c", subcore_axis_name="s")
    f = pl.kernel(
        _sc_gather_body,
        out_type=(jax.ShapeDtypeStruct((N_EDGES, D), jnp.float32),
                  jax.ShapeDtypeStruct((N_EDGES, D), jnp.float32)),
        mesh=mesh,
        scratch_types=[
            pltpu.VMEM((G_CHUNK,), jnp.int32),
            pltpu.VMEM((G_CHUNK, D), jnp.float32),
            pltpu.VMEM((G_TAIL,), jnp.int32),
            pltpu.VMEM((G_TAIL, D), jnp.float32),
            pltpu.SemaphoreType.DMA,
        ],
    )
    return f(table, src_idx, tgt_idx)


S_CHUNK = 1280
S_STEPS = N_EDGES // S_CHUNK
ACC_COLS = 5120
CW = 16
OHP = ACC_COLS


def _splat(v, j):
    idx = jnp.full((16,), j, dtype=jnp.int32)
    return lax.gather(
        v, idx[:, None],
        lax.GatherDimensionNumbers(offset_dims=(), collapsed_slice_dims=(0,),
                                   start_index_map=(0,)),
        (1,), mode=lax.GatherScatterMode.PROMISE_IN_BOUNDS)


def _sc_scatter_body(msgT_hbm, tgt_hbm, zeros_hbm, out_hbm,
                     acc, idx_v, buf, sem):
    c = lax.axis_index("c")
    s = lax.axis_index("s")
    pltpu.sync_copy(zeros_hbm, acc)
    nbase = c * HALF
    iota = lax.iota(jnp.int32, 16)
    dummy = HALF + (iota & 7)
    rowoff = iota * S_CHUNK
    accoff = iota * ACC_COLS

    def step(k, carry):
        off = k * S_CHUNK
        pltpu.sync_copy(tgt_hbm.at[pl.ds(off, S_CHUNK)], idx_v)
        copies = [
            pltpu.make_async_copy(
                msgT_hbm.at[s * CW + f, pl.ds(off, S_CHUNK)],
                buf.at[pl.ds(f * S_CHUNK, S_CHUNK)], sem)
            for f in range(CW)
        ]
        for d in copies:
            d.start()
        for d in copies:
            d.wait()

        def grp(g, carry2):
            tv = idx_v[pl.ds(g * 16, 16)]
            lv = tv - nbase
            ok = (lv >= 0) & (lv < HALF)
            lv = jnp.where(ok, lv, dummy)
            for j in range(16):
                col = jnp.broadcast_to(g * 16 + j, (16,)).astype(jnp.int32)
                val = plsc.load_gather(buf, [rowoff + col])
                plsc.addupdate_scatter(acc, [accoff + _splat(lv, j)], val)
            return carry2

        lax.fori_loop(0, S_CHUNK // 16, grp, 0)
        return carry

    lax.fori_loop(0, S_STEPS, step, 0)
    for f in range(CW):
        pltpu.sync_copy(acc.at[pl.ds(f * ACC_COLS, ACC_COLS)],
                        out_hbm.at[s * CW + f, pl.ds(c * OHP, ACC_COLS)])


@jax.jit
def _sc_scatter(msgT, tgt_idx, zeros):
    mesh = plsc.VectorSubcoreMesh(core_axis_name="c", subcore_axis_name="s")
    f = pl.kernel(
        _sc_scatter_body,
        out_type=jax.ShapeDtypeStruct((D, 2 * OHP), jnp.float32),
        mesh=mesh,
        compiler_params=pltpu.CompilerParams(needs_layout_passes=False),
        scratch_types=[
            pltpu.VMEM((CW * ACC_COLS,), jnp.float32),
            pltpu.VMEM((S_CHUNK,), jnp.int32),
            pltpu.VMEM((CW * S_CHUNK,), jnp.float32),
            pltpu.SemaphoreType.DMA,
        ],
    )
    o = f(msgT, tgt_idx, zeros)
    return jnp.concatenate([o[:, :HALF], o[:, OHP:OHP + HALF]], axis=1).T


BN_BLK = 1000
HI = jax.lax.Precision.HIGHEST


def _node0_body(types_ref, emb_ref, s_ref, t_ref, state_out, abn_out):
    ty = types_ref[:, 0]
    oh = (ty[:, None] == lax.broadcasted_iota(jnp.int32, (BN_BLK, ATOM_CLASSES), 1)
          ).astype(jnp.float32)
    st = jnp.dot(oh, emb_ref[...], precision=HI)
    state_out[...] = st
    abn_out[...] = st * s_ref[...] + t_ref[...]


def _noden_body(prev_ref, agg_ref, s_ref, t_ref, state_out, abn_out):
    st = prev_ref[...] + agg_ref[...]
    state_out[...] = st
    abn_out[...] = st * s_ref[...] + t_ref[...]


def _node0(atom_types2, atom_emb, s_vec, t_vec):
    grid = N_NODES // BN_BLK
    return pl.pallas_call(
        _node0_body,
        grid=(grid,),
        in_specs=[
            pl.BlockSpec((BN_BLK, 1), lambda i: (i, 0)),
            pl.BlockSpec((ATOM_CLASSES, D), lambda i: (0, 0)),
            pl.BlockSpec((1, D), lambda i: (0, 0)),
            pl.BlockSpec((1, D), lambda i: (0, 0)),
        ],
        out_specs=[
            pl.BlockSpec((BN_BLK, D), lambda i: (i, 0)),
            pl.BlockSpec((BN_BLK, D), lambda i: (i, 0)),
        ],
        out_shape=[jax.ShapeDtypeStruct((N_NODES, D), jnp.float32)] * 2,
    )(atom_types2, atom_emb, s_vec, t_vec)


def _noden(prev, agg, s_vec, t_vec):
    grid = N_NODES // BN_BLK
    return pl.pallas_call(
        _noden_body,
        grid=(grid,),
        in_specs=[
            pl.BlockSpec((BN_BLK, D), lambda i: (i, 0)),
            pl.BlockSpec((BN_BLK, D), lambda i: (i, 0)),
            pl.BlockSpec((1, D), lambda i: (0, 0)),
            pl.BlockSpec((1, D), lambda i: (0, 0)),
        ],
        out_specs=[
            pl.BlockSpec((BN_BLK, D), lambda i: (i, 0)),
            pl.BlockSpec((BN_BLK, D), lambda i: (i, 0)),
        ],
        out_shape=[jax.ShapeDtypeStruct((N_NODES, D), jnp.float32)] * 2,
    )(prev, agg, s_vec, t_vec)


BE_BLK = 640


def _edge_core(braw, src_ref, tgt_ref, w1s, w1t, w1b, w2, b2, au, sb, tb,
               msg_out, bond_out):
    sf = src_ref[...].astype(jnp.bfloat16)
    tf = tgt_ref[...].astype(jnp.bfloat16)
    bbn = (braw * sb[...] + tb[...]).astype(jnp.bfloat16)
    acc = (jnp.dot(sf, w1s[...], preferred_element_type=jnp.float32)
           + jnp.dot(tf, w1t[...], preferred_element_type=jnp.float32)
           + jnp.dot(bbn, w1b[...], preferred_element_type=jnp.float32))
    h = jax.nn.sigmoid(acc).astype(jnp.bfloat16)
    nb = jnp.dot(h, w2[...], preferred_element_type=jnp.float32) + b2[...]
    u = jax.nn.sigmoid(jnp.dot(sf, au[...], preferred_element_type=jnp.float32))
    msg_out[...] = (u * nb).T
    if bond_out is not None:
        bond_out[...] = braw + nb


def _edge0_body(btypes_ref, bemb_ref, src_ref, tgt_ref, w1s, w1t, w1b, w2, b2,
                au, sb, tb, msg_out, bond_out):
    ty = btypes_ref[:, 0]
    oh = (ty[:, None] == lax.broadcasted_iota(jnp.int32, (BE_BLK, BOND_CLASSES), 1)
          ).astype(jnp.float32)
    braw = jnp.dot(oh, bemb_ref[...], precision=HI)
    _edge_core(braw, src_ref, tgt_ref, w1s, w1t, w1b, w2, b2, au, sb, tb,
               msg_out, bond_out)


def _edgen_body(bond_ref, src_ref, tgt_ref, w1s, w1t, w1b, w2, b2, au, sb, tb,
                msg_out, bond_out):
    _edge_core(bond_ref[...], src_ref, tgt_ref, w1s, w1t, w1b, w2, b2, au, sb,
               tb, msg_out, bond_out)


def _edge_wspecs():
    return [
        pl.BlockSpec((D, 2 * D), lambda i: (0, 0)),
        pl.BlockSpec((D, 2 * D), lambda i: (0, 0)),
        pl.BlockSpec((D, 2 * D), lambda i: (0, 0)),
        pl.BlockSpec((2 * D, D), lambda i: (0, 0)),
        pl.BlockSpec((1, D), lambda i: (0, 0)),
        pl.BlockSpec((D, D), lambda i: (0, 0)),
        pl.BlockSpec((1, D), lambda i: (0, 0)),
        pl.BlockSpec((1, D), lambda i: (0, 0)),
    ]


def _final_body(state_ref, agg_ref, ngi_ref, w0, b0, w1, b1, wl, bl,
                out_ref, acc):
    i = pl.program_id(0)

    @pl.when(i == 0)
    def _():
        acc[...] = jnp.zeros_like(acc)

    st = state_ref[...] + agg_ref[...]
    g = ngi_ref[:, 0]
    oh = (g[:, None] == lax.broadcasted_iota(jnp.int32, (BN_BLK, N_GRAPHS), 1)
          ).astype(jnp.float32)
    acc[...] += lax.dot_general(oh, st, (((0,), (0,)), ((), ())), precision=HI)

    @pl.when(i == pl.num_programs(0) - 1)
    def _():
        mol = acc[...]
        m = jax.nn.relu(jnp.dot(mol, w0[...], precision=HI) + b0[...])
        m = jax.nn.relu(jnp.dot(m, w1[...], precision=HI) + b1[...])
        out_ref[...] = jnp.dot(m, wl[...], precision=HI) + bl[...]


def _final(state, agg, ngi2, w0, b0, w1, b1, wl, bl):
    grid = N_NODES // BN_BLK
    return pl.pallas_call(
        _final_body,
        grid=(grid,),
        in_specs=[
            pl.BlockSpec((BN_BLK, D), lambda i: (i, 0)),
            pl.BlockSpec((BN_BLK, D), lambda i: (i, 0)),
            pl.BlockSpec((BN_BLK, 1), lambda i: (i, 0)),
            pl.BlockSpec((D, 256), lambda i: (0, 0)),
            pl.BlockSpec((1, 256), lambda i: (0, 0)),
            pl.BlockSpec((256, 128), lambda i: (0, 0)),
            pl.BlockSpec((1, 128), lambda i: (0, 0)),
            pl.BlockSpec((128, 1), lambda i: (0, 0)),
            pl.BlockSpec((1, 1), lambda i: (0, 0)),
        ],
        out_specs=pl.BlockSpec((N_GRAPHS, 1), lambda i: (0, 0)),
        out_shape=jax.ShapeDtypeStruct((N_GRAPHS, 1), jnp.float32),
        scratch_shapes=[pltpu.VMEM((N_GRAPHS, D), jnp.float32)],
    )(state, agg, ngi2, w0, b0, w1, b1, wl, bl)


def kernel(atom_types, bond_types, node_graph_indices, connectivity, atom_emb,
           bond_emb, atom_bn_gamma, atom_bn_beta, atom_bn_mean, atom_bn_var,
           bond_bn_gamma, bond_bn_beta, bond_bn_mean, bond_bn_var, bu1_W,
           bu2_W, bu2_b, au_W, out0_W, out0_b, out1_W, out1_b, last_W, last_b):
    eps = 1e-3
    sa = atom_bn_gamma / jnp.sqrt(atom_bn_var + eps)
    ta = atom_bn_beta - atom_bn_mean * sa
    sb = bond_bn_gamma / jnp.sqrt(bond_bn_var + eps)
    tb = bond_bn_beta - bond_bn_mean * sb

    tgt_idx = connectivity[:, 0]
    src_idx = connectivity[:, 1]
    zeros = jnp.zeros((CW * ACC_COLS,), jnp.float32)

    at2 = atom_types.reshape(N_NODES, 1)
    bt2 = bond_types.reshape(N_EDGES, 1)
    ngi2 = node_graph_indices.reshape(N_NODES, 1)

    w1s = [bu1_W[i, :D].astype(jnp.bfloat16) for i in range(L)]
    w1t = [bu1_W[i, D:2 * D].astype(jnp.bfloat16) for i in range(L)]
    w1b = [bu1_W[i, 2 * D:].astype(jnp.bfloat16) for i in range(L)]
    w2 = [bu2_W[i].astype(jnp.bfloat16) for i in range(L)]
    au = [au_W[i].astype(jnp.bfloat16) for i in range(L)]
    b2 = [bu2_b[i].reshape(1, D) for i in range(L)]
    sbv = [sb[i].reshape(1, D) for i in range(L)]
    tbv = [tb[i].reshape(1, D) for i in range(L)]
    sav = [sa[i].reshape(1, D) for i in range(L)]
    tav = [ta[i].reshape(1, D) for i in range(L)]

    eb = pl.BlockSpec((BE_BLK, D), lambda i: (i, 0))
    grid_e = N_EDGES // BE_BLK

    state, abn = _node0(at2, atom_emb, sav[0], tav[0])
    bond = None
    agg = None
    for i in range(L):
        if i > 0:
            state, abn = _noden(state, agg, sav[i], tav[i])
        src_rows, tgt_rows = _sc_gather(abn, src_idx, tgt_idx)
        first = (i == 0)
        want_bond = (i < L - 1)
        if first:
            lead_args = [bt2, bond_emb]
            lead_specs = [pl.BlockSpec((BE_BLK, 1), lambda i: (i, 0)),
                          pl.BlockSpec((BOND_CLASSES, D), lambda i: (0, 0))]
            body = _edge0_body
        else:
            lead_args = [bond]
            lead_specs = [eb]
            body = _edgen_body
        ebT = pl.BlockSpec((D, BE_BLK), lambda i: (0, i))
        if want_bond:
            fn = body
            out_specs = [ebT, eb]
            out_shape = [jax.ShapeDtypeStruct((D, N_EDGES), jnp.float32),
                         jax.ShapeDtypeStruct((N_EDGES, D), jnp.float32)]
        else:
            def fn(*args, _body=body):
                _body(*args, None)
            out_specs = [ebT]
            out_shape = [jax.ShapeDtypeStruct((D, N_EDGES), jnp.float32)]
        res = pl.pallas_call(
            fn,
            grid=(grid_e,),
            in_specs=lead_specs + [eb, eb] + _edge_wspecs(),
            out_specs=out_specs,
            out_shape=out_shape,
        )(*lead_args, src_rows, tgt_rows, w1s[i], w1t[i], w1b[i], w2[i],
          b2[i], au[i], sbv[i], tbv[i])
        if want_bond:
            msg, bond = res
        else:
            msg, = res
        agg = _sc_scatter(msg, tgt_idx, zeros)

    return _final(state, agg, ngi2,
                  out0_W, out0_b.reshape(1, 256),
                  out1_W, out1_b.reshape(1, 128),
                  last_W, last_b.reshape(1, 1))
